# Initial kernel scaffold; baseline (speedup 1.0000x reference)
import jax
import jax.numpy as jnp
from jax.experimental import pallas as pl


def kernel(state, action, x, edge_index, batch, g1w1, g1b1, g1w2, g1b2, g2w1, g2b1, g2w2, g2b2, q1w1, q1b1, q1w2, q1b2, q1w3, q1b3, q2w1, q2b1, q2w2, q2b2, q2w3, q2b3):
    raise NotImplementedError("write your pallas kernel here")


# speedup vs baseline: 2.5829x; 2.5829x over previous
import functools

import jax
import jax.numpy as jnp
from jax import lax
from jax.experimental import pallas as pl
from jax.experimental.pallas import tpu as pltpu
from jax.experimental.pallas import tpu_sc as plsc

_F32 = jnp.float32
_KCH = 64


def _dot(a, b):
    return lax.dot_general(a, b, (((1,), (1,)), ((), ())),
                           preferred_element_type=_F32,
                           precision=lax.Precision.HIGHEST)


def _tc1_body(nf_ref, wd_ref, ws_ref, b1_ref, a_ref, b_ref):
    nf = nf_ref[...]
    a_ref[...] = _dot(nf, wd_ref[...]) + b1_ref[...]
    b_ref[...] = _dot(nf, ws_ref[...])


def _tc1(nf, wd, ws, b1, Rb):
    Npad, ND = nf.shape
    GH = wd.shape[0]
    grid = (Npad // Rb,)
    return pl.pallas_call(
        _tc1_body,
        grid=grid,
        in_specs=[
            pl.BlockSpec((Rb, ND), lambda i: (i, 0)),
            pl.BlockSpec((GH, ND), lambda i: (0, 0)),
            pl.BlockSpec((GH, ND), lambda i: (0, 0)),
            pl.BlockSpec((1, GH), lambda i: (0, 0)),
        ],
        out_specs=[
            pl.BlockSpec((Rb, GH), lambda i: (i, 0)),
            pl.BlockSpec((Rb, GH), lambda i: (i, 0)),
        ],
        out_shape=[
            jax.ShapeDtypeStruct((Npad, GH), _F32),
            jax.ShapeDtypeStruct((Npad, GH), _F32),
        ],
        compiler_params=pltpu.CompilerParams(
            dimension_semantics=("arbitrary",)),
    )(nf, wd, ws, b1)


def _tc2_body(acc_ref, cnt_ref, w2_ref, b2_ref, wd2_ref, ws2_ref, b21_ref,
              ab_ref):
    S = acc_ref[0] + acc_ref[1]
    cnt = cnt_ref[0, :, 0:1] + cnt_ref[1, :, 0:1]
    mask = (cnt > 0.0).astype(_F32)
    mean = S * (1.0 / jnp.maximum(cnt, 1.0))
    h = jnp.maximum(_dot(mean, w2_ref[...]) + b2_ref[...] * mask, 0.0)
    ab_ref[...] = jnp.concatenate(
        [_dot(h, wd2_ref[...]) + b21_ref[...], _dot(h, ws2_ref[...])],
        axis=1)


def _tc2(acc, cnt, w2, b2, wd2, ws2, b21, Rb):
    _, Npad, GH = acc.shape
    G = wd2.shape[0]
    grid = (Npad // Rb,)
    return pl.pallas_call(
        _tc2_body,
        grid=grid,
        in_specs=[
            pl.BlockSpec((2, Rb, GH), lambda i: (0, i, 0)),
            pl.BlockSpec((2, Rb, 128), lambda i: (0, i, 0)),
            pl.BlockSpec((GH, GH), lambda i: (0, 0)),
            pl.BlockSpec((1, GH), lambda i: (0, 0)),
            pl.BlockSpec((G, GH), lambda i: (0, 0)),
            pl.BlockSpec((G, GH), lambda i: (0, 0)),
            pl.BlockSpec((1, G), lambda i: (0, 0)),
        ],
        out_specs=pl.BlockSpec((Rb, 2 * G), lambda i: (i, 0)),
        out_shape=jax.ShapeDtypeStruct((Npad, 2 * G), _F32),
        compiler_params=pltpu.CompilerParams(
            dimension_semantics=("arbitrary",)),
    )(acc, cnt, w2, b2, wd2, ws2, b21)


def _make_tc3_body(B, G):
    def body(acc2_ref, cnt_ref, bt_ref, st_ref, ac_ref, g2w2_ref, g2b2_ref,
             q1w1_ref, q1b1_ref, q1w2_ref, q1b2_ref, q1w3_ref, q1b3_ref,
             q2w1_ref, q2b1_ref, q2w2_ref, q2b2_ref, q2w3_ref, q2b3_ref,
             q1_ref, q2_ref, accP):
        i = pl.program_id(0)
        nsteps = pl.num_programs(0)

        @pl.when(i == 0)
        def _():
            accP[...] = jnp.zeros_like(accP)

        S2 = (acc2_ref[0] + acc2_ref[1])[:, 0:G]
        cnt = cnt_ref[0, :, 0:1] + cnt_ref[1, :, 0:1]
        mask = (cnt > 0.0).astype(_F32)
        P = S2 * (1.0 / jnp.maximum(cnt, 1.0))
        Rb = P.shape[0]
        ones = jnp.ones((Rb, 1), _F32)
        zrest = jnp.zeros((Rb, 128 - G - 2), _F32)
        P2 = jnp.concatenate([P, mask, ones, zrest], axis=1)
        bt = bt_ref[0]
        iot = lax.broadcasted_iota(jnp.int32, (128, Rb), 0)
        ohT = (iot == bt).astype(_F32)
        accP[...] += lax.dot_general(ohT, P2, (((1,), (0,)), ((), ())),
                                     preferred_element_type=_F32,
                                     precision=lax.Precision.HIGHEST)

        @pl.when(i == nsteps - 1)
        def _():
            A = accP[...]
            geS = A[:, 0:G]
            gm = A[:, G:G + 1]
            gn = A[:, G + 1:G + 2]
            ge = (_dot(geS, g2w2_ref[...]) + g2b2_ref[...] * gm) * (
                1.0 / jnp.maximum(gn, 1.0))
            z = jnp.concatenate([st_ref[...], ac_ref[...], ge], axis=1)
            h1 = jnp.maximum(_dot(z, q1w1_ref[...]) + q1b1_ref[...], 0.0)
            h1 = jnp.maximum(_dot(h1, q1w2_ref[...]) + q1b2_ref[...], 0.0)
            q1 = _dot(h1, q1w3_ref[...])[:, 0:1] + q1b3_ref[0, 0]
            h2 = jnp.maximum(_dot(z, q2w1_ref[...]) + q2b1_ref[...], 0.0)
            h2 = jnp.maximum(_dot(h2, q2w2_ref[...]) + q2b2_ref[...], 0.0)
            q2 = _dot(h2, q2w3_ref[...])[:, 0:1] + q2b3_ref[0, 0]
            q1_ref[...] = q1[0:B, :]
            q2_ref[...] = q2[0:B, :]

    return body


def _tc3(acc2, cnt, bt3, st, ac, g2w2, g2b2, qw, Rb, Bq):
    _, Npad, _ = acc2.shape
    G = g2w2.shape[0]
    H = qw[0].shape[0]
    QIN = qw[0].shape[1]
    grid = (Npad // Rb,)
    body = _make_tc3_body(Bq, G)
    wspecs = [
        pl.BlockSpec((H, QIN), lambda i: (0, 0)),
        pl.BlockSpec((1, H), lambda i: (0, 0)),
        pl.BlockSpec((H, H), lambda i: (0, 0)),
        pl.BlockSpec((1, H), lambda i: (0, 0)),
        pl.BlockSpec((128, H), lambda i: (0, 0)),
        pl.BlockSpec((1, 1), lambda i: (0, 0)),
    ]
    return pl.pallas_call(
        body,
        grid=grid,
        in_specs=[
            pl.BlockSpec((2, Rb, 128), lambda i: (0, i, 0)),
            pl.BlockSpec((2, Rb, 128), lambda i: (0, i, 0)),
            pl.BlockSpec((1, 1, Rb), lambda i: (i, 0, 0)),
            pl.BlockSpec((st.shape[0], st.shape[1]), lambda i: (0, 0)),
            pl.BlockSpec((ac.shape[0], ac.shape[1]), lambda i: (0, 0)),
            pl.BlockSpec((G, G), lambda i: (0, 0)),
            pl.BlockSpec((1, G), lambda i: (0, 0)),
        ] + wspecs + wspecs,
        out_specs=[
            pl.BlockSpec((Bq, 1), lambda i: (0, 0)),
            pl.BlockSpec((Bq, 1), lambda i: (0, 0)),
        ],
        out_shape=[
            jax.ShapeDtypeStruct((Bq, 1), _F32),
            jax.ShapeDtypeStruct((Bq, 1), _F32),
        ],
        scratch_shapes=[pltpu.VMEM((128, 128), _F32)],
        compiler_params=pltpu.CompilerParams(
            dimension_semantics=("arbitrary",)),
    )(acc2, cnt, bt3, st, ac, g2w2, g2b2, *qw)


def _sc_edge(a_nodes, b_nodes, ei, n_nodes, K, packed):
    Npad, Dw = a_nodes.shape
    D = Dw
    Gc = (Dw // 2 if packed else Dw) // 16
    G = D // 16
    Epad = ei.shape[1]
    mesh = plsc.VectorSubcoreMesh(core_axis_name="c", subcore_axis_name="s")
    NC, NS = mesh.num_cores, mesh.num_subcores
    NW = NC * NS
    ALGN = NS * 8
    NROW = min(Npad, ((n_nodes + 1 + ALGN - 1) // ALGN) * ALGN)
    NCHW = Epad // (K * NW)

    out_type = jax.ShapeDtypeStruct((NC, Npad, D), _F32)
    scratch = [
        pltpu.VMEM((K,), jnp.int32),
        pltpu.VMEM((K,), jnp.int32),
        pltpu.VMEM((K, Dw), _F32),
        pltpu.VMEM((K, Dw), _F32),
        pltpu.VMEM_SHARED((NROW, D), _F32),
        pltpu.SemaphoreType.DMA,
        pltpu.SemaphoreType.DMA,
    ]
    if packed:
        scratch.append(pltpu.VMEM((K, D), _F32))

    def body(a_hbm, b_hbm, ei_hbm, acc_out, *rest):
        rest = list(rest)
        idxs_v, idxd_v, ga_v, gb_v, acc_sh, sem1, sem2 = rest[:7]
        gr_v = rest[7] if packed else ga_v
        cid = lax.axis_index("c")
        sid = lax.axis_index("s")
        wid = sid * NC + cid
        zv = jnp.zeros((16,), _F32)

        def zrow(i, carry):
            for j in range(G):
                gr_v[i, pl.ds(j * 16, 16)] = zv
            return carry

        lax.fori_loop(0, K, zrow, 0)

        RPW = NROW // NS
        nfull, rem = RPW // K, RPW % K
        base = sid * RPW
        for k in range(nfull):
            pltpu.sync_copy(gr_v, acc_sh.at[pl.ds(base + k * K, K)])
        if rem:
            pltpu.sync_copy(gr_v.at[pl.ds(0, rem)],
                            acc_sh.at[pl.ds(base + nfull * K, rem)])

        TAIL = Npad - NROW

        @pl.when(sid == 0)
        def _():
            t = 0
            while t < TAIL:
                c = min(K, TAIL - t)
                pltpu.sync_copy(gr_v.at[pl.ds(0, c)],
                                acc_out.at[cid, pl.ds(NROW + t, c)])
                t += c

        plsc.subcore_barrier()

        def chunk(t, carry):
            off = (wid * NCHW + t) * K
            pltpu.sync_copy(ei_hbm.at[0, pl.ds(off, K)], idxs_v)
            pltpu.sync_copy(ei_hbm.at[1, pl.ds(off, K)], idxd_v)
            cp1 = pltpu.async_copy(a_hbm.at[idxd_v], ga_v, sem1)
            cp2 = pltpu.async_copy(b_hbm.at[idxs_v], gb_v, sem2)
            cp1.wait()
            cp2.wait()

            def crow(i, c2):
                for j in range(Gc):
                    s = pl.ds(j * 16, 16)
                    sb = pl.ds(Dw // 2 + j * 16, 16) if packed else s
                    gr_v[i, s] = jnp.maximum(ga_v[i, s] + gb_v[i, sb], 0.0)
                return c2

            lax.fori_loop(0, K, crow, 0, unroll=2)
            pltpu.sync_copy(gr_v, acc_sh.at[idxd_v], add=True)
            return carry

        lax.fori_loop(0, NCHW, chunk, 0)
        plsc.subcore_barrier()

        for k in range(nfull):
            r = base + k * K
            pltpu.sync_copy(acc_sh.at[pl.ds(r, K)],
                            acc_out.at[cid, pl.ds(r, K)])
        if rem:
            r = base + nfull * K
            pltpu.sync_copy(acc_sh.at[pl.ds(r, rem)],
                            acc_out.at[cid, pl.ds(r, rem)])

    fn = pl.kernel(body, out_type=out_type, mesh=mesh,
                   scratch_types=scratch)
    return fn(a_nodes, b_nodes, ei)


def _sc_count(ei, n_nodes, Npad, K):
    D = 128
    G = D // 16
    Epad = ei.shape[1]
    mesh = plsc.VectorSubcoreMesh(core_axis_name="c", subcore_axis_name="s")
    NC, NS = mesh.num_cores, mesh.num_subcores
    NW = NC * NS
    ALGN = NS * 8
    NROW = min(Npad, ((n_nodes + 1 + ALGN - 1) // ALGN) * ALGN)
    NCHW = Epad // (K * NW)

    def body(ei_hbm, cnt_out, idxd_v, ones_v, cnt_sh):
        cid = lax.axis_index("c")
        sid = lax.axis_index("s")
        wid = sid * NC + cid
        zv = jnp.zeros((16,), _F32)
        ov = jnp.ones((16,), _F32)

        def zrow(i, carry):
            for j in range(G):
                ones_v[i, pl.ds(j * 16, 16)] = zv
            return carry

        lax.fori_loop(0, K, zrow, 0)

        RPW = NROW // NS
        nfull, rem = RPW // K, RPW % K
        base = sid * RPW
        for k in range(nfull):
            pltpu.sync_copy(ones_v, cnt_sh.at[pl.ds(base + k * K, K)])
        if rem:
            pltpu.sync_copy(ones_v.at[pl.ds(0, rem)],
                            cnt_sh.at[pl.ds(base + nfull * K, rem)])
        TAIL = Npad - NROW

        @pl.when(sid == 0)
        def _():
            t = 0
            while t < TAIL:
                c = min(K, TAIL - t)
                pltpu.sync_copy(ones_v.at[pl.ds(0, c)],
                                cnt_out.at[cid, pl.ds(NROW + t, c)])
                t += c

        def orow(i, carry):
            for j in range(G):
                ones_v[i, pl.ds(j * 16, 16)] = ov
            return carry

        lax.fori_loop(0, K, orow, 0)
        plsc.subcore_barrier()

        def chunk(t, carry):
            off = (wid * NCHW + t) * K
            pltpu.sync_copy(ei_hbm.at[1, pl.ds(off, K)], idxd_v)
            pltpu.sync_copy(ones_v, cnt_sh.at[idxd_v], add=True)
            return carry

        lax.fori_loop(0, NCHW, chunk, 0)
        plsc.subcore_barrier()

        for k in range(nfull):
            r = base + k * K
            pltpu.sync_copy(cnt_sh.at[pl.ds(r, K)],
                            cnt_out.at[cid, pl.ds(r, K)])
        if rem:
            r = base + nfull * K
            pltpu.sync_copy(cnt_sh.at[pl.ds(r, rem)],
                            cnt_out.at[cid, pl.ds(r, rem)])

    fn = pl.kernel(
        body,
        out_type=jax.ShapeDtypeStruct((NC, Npad, D), _F32),
        mesh=mesh,
        scratch_types=[
            pltpu.VMEM((K,), jnp.int32),
            pltpu.VMEM((K, D), _F32),
            pltpu.VMEM_SHARED((NROW, D), _F32),
        ],
    )
    return fn(ei)


def kernel(state, action, x, edge_index, batch, g1w1, g1b1, g1w2, g1b2,
           g2w1, g2b1, g2w2, g2b2, q1w1, q1b1, q1w2, q1b2, q1w3, q1b3,
           q2w1, q2b1, q2w2, q2b2, q2w3, q2b3):
    N, SD = x.shape
    B, AD = action.shape
    E = edge_index.shape[1]
    ND = SD + AD
    GH = g1w2.shape[0]
    G = g2w2.shape[0]
    rep = N // B

    Rb = 512
    ALIGN = 2048
    Npad = ((N + 1 + ALIGN - 1) // ALIGN) * ALIGN
    CHW = _KCH * 32
    Epad = ((E + CHW - 1) // CHW) * CHW

    nf = jnp.concatenate([x, jnp.repeat(action, rep, axis=0)], axis=1)
    nf = jnp.pad(nf, ((0, Npad - N), (0, 0)))
    ei = edge_index
    if Epad != E:
        ei = jnp.concatenate(
            [ei, jnp.full((2, Epad - E), N, edge_index.dtype)], axis=1)

    cnt = _sc_count(ei, N, Npad, _KCH)
    A1, B1 = _tc1(nf, g1w1[:, :ND], g1w1[:, ND:], g1b1.reshape(1, -1), Rb)
    acc1 = _sc_edge(A1, B1, ei, N, K=_KCH, packed=False)
    AB2 = _tc2(acc1, cnt, g1w2, g1b2.reshape(1, -1),
               g2w1[:, :GH], g2w1[:, GH:], g2b1.reshape(1, -1), Rb)
    acc2 = _sc_edge(AB2, AB2, ei, N, K=_KCH, packed=True)

    bt = jnp.pad(batch.astype(jnp.int32), (0, Npad - N),
                 constant_values=min(127, B))
    bt3 = bt.reshape(Npad // Rb, 1, Rb)
    stp = jnp.pad(state, ((0, 128 - B), (0, 0)))
    acp = jnp.pad(action, ((0, 128 - B), (0, 0)))
    q1w3p = jnp.pad(q1w3, ((0, 128 - q1w3.shape[0]), (0, 0)))
    q2w3p = jnp.pad(q2w3, ((0, 128 - q2w3.shape[0]), (0, 0)))
    qw = [q1w1, q1b1.reshape(1, -1), q1w2, q1b2.reshape(1, -1),
          q1w3p, q1b3.reshape(1, -1),
          q2w1, q2b1.reshape(1, -1), q2w2, q2b2.reshape(1, -1),
          q2w3p, q2b3.reshape(1, -1)]
    q1, q2 = _tc3(acc2, cnt, bt3, stp, acp, g2w2, g2b2.reshape(1, -1),
                  qw, Rb, B)
    return (q1, q2)


# speedup vs baseline: 2.9241x; 1.1321x over previous
import functools

import jax
import jax.numpy as jnp
from jax import lax
from jax.experimental import pallas as pl
from jax.experimental.pallas import tpu as pltpu
from jax.experimental.pallas import tpu_sc as plsc

_F32 = jnp.float32
_KCH = 64


def _dot(a, b):
    return lax.dot_general(a, b, (((1,), (1,)), ((), ())),
                           preferred_element_type=_F32,
                           precision=lax.Precision.HIGHEST)


def _tc1_body(nf_ref, wd_ref, ws_ref, b1_ref, a_ref, b_ref):
    nf = nf_ref[...]
    a_ref[...] = _dot(nf, wd_ref[...]) + b1_ref[...]
    b_ref[...] = _dot(nf, ws_ref[...])


def _tc1(nf, wd, ws, b1, Rb):
    Npad, ND = nf.shape
    GH = wd.shape[0]
    grid = (Npad // Rb,)
    return pl.pallas_call(
        _tc1_body,
        grid=grid,
        in_specs=[
            pl.BlockSpec((Rb, ND), lambda i: (i, 0)),
            pl.BlockSpec((GH, ND), lambda i: (0, 0)),
            pl.BlockSpec((GH, ND), lambda i: (0, 0)),
            pl.BlockSpec((1, GH), lambda i: (0, 0)),
        ],
        out_specs=[
            pl.BlockSpec((Rb, GH), lambda i: (i, 0)),
            pl.BlockSpec((Rb, GH), lambda i: (i, 0)),
        ],
        out_shape=[
            jax.ShapeDtypeStruct((Npad, GH), _F32),
            jax.ShapeDtypeStruct((Npad, GH), _F32),
        ],
        compiler_params=pltpu.CompilerParams(
            dimension_semantics=("arbitrary",)),
    )(nf, wd, ws, b1)


def _tc2_body(acc_ref, cnt_ref, w2_ref, b2_ref, wd2_ref, ws2_ref, b21_ref,
              ab_ref):
    S = acc_ref[0] + acc_ref[1]
    cnt = cnt_ref[0, :, 0:1] + cnt_ref[1, :, 0:1]
    mask = (cnt > 0.0).astype(_F32)
    mean = S * (1.0 / jnp.maximum(cnt, 1.0))
    h = jnp.maximum(_dot(mean, w2_ref[...]) + b2_ref[...] * mask, 0.0)
    ab_ref[...] = jnp.concatenate(
        [_dot(h, wd2_ref[...]) + b21_ref[...], _dot(h, ws2_ref[...])],
        axis=1)


def _tc2(acc, cnt, w2, b2, wd2, ws2, b21, Rb):
    _, Npad, GH = acc.shape
    G = wd2.shape[0]
    grid = (Npad // Rb,)
    return pl.pallas_call(
        _tc2_body,
        grid=grid,
        in_specs=[
            pl.BlockSpec((2, Rb, GH), lambda i: (0, i, 0)),
            pl.BlockSpec((2, Rb, 128), lambda i: (0, i, 0)),
            pl.BlockSpec((GH, GH), lambda i: (0, 0)),
            pl.BlockSpec((1, GH), lambda i: (0, 0)),
            pl.BlockSpec((G, GH), lambda i: (0, 0)),
            pl.BlockSpec((G, GH), lambda i: (0, 0)),
            pl.BlockSpec((1, G), lambda i: (0, 0)),
        ],
        out_specs=pl.BlockSpec((Rb, 2 * G), lambda i: (i, 0)),
        out_shape=jax.ShapeDtypeStruct((Npad, 2 * G), _F32),
        compiler_params=pltpu.CompilerParams(
            dimension_semantics=("arbitrary",)),
    )(acc, cnt, w2, b2, wd2, ws2, b21)


def _make_tc3_body(B, G):
    def body(acc2_ref, cnt_ref, bt_ref, st_ref, ac_ref, g2w2_ref, g2b2_ref,
             q1w1_ref, q1b1_ref, q1w2_ref, q1b2_ref, q1w3_ref, q1b3_ref,
             q2w1_ref, q2b1_ref, q2w2_ref, q2b2_ref, q2w3_ref, q2b3_ref,
             q1_ref, q2_ref, accP):
        i = pl.program_id(0)
        nsteps = pl.num_programs(0)

        @pl.when(i == 0)
        def _():
            accP[...] = jnp.zeros_like(accP)

        S2 = (acc2_ref[0] + acc2_ref[1])[:, 0:G]
        cnt = cnt_ref[0, :, 0:1] + cnt_ref[1, :, 0:1]
        mask = (cnt > 0.0).astype(_F32)
        P = S2 * (1.0 / jnp.maximum(cnt, 1.0))
        Rb = P.shape[0]
        ones = jnp.ones((Rb, 1), _F32)
        zrest = jnp.zeros((Rb, 128 - G - 2), _F32)
        P2 = jnp.concatenate([P, mask, ones, zrest], axis=1)
        bt = bt_ref[0]
        iot = lax.broadcasted_iota(jnp.int32, (128, Rb), 0)
        ohT = (iot == bt).astype(_F32)
        accP[...] += lax.dot_general(ohT, P2, (((1,), (0,)), ((), ())),
                                     preferred_element_type=_F32,
                                     precision=lax.Precision.HIGHEST)

        @pl.when(i == nsteps - 1)
        def _():
            A = accP[...]
            geS = A[:, 0:G]
            gm = A[:, G:G + 1]
            gn = A[:, G + 1:G + 2]
            ge = (_dot(geS, g2w2_ref[...]) + g2b2_ref[...] * gm) * (
                1.0 / jnp.maximum(gn, 1.0))
            z = jnp.concatenate([st_ref[...], ac_ref[...], ge], axis=1)
            h1 = jnp.maximum(_dot(z, q1w1_ref[...]) + q1b1_ref[...], 0.0)
            h1 = jnp.maximum(_dot(h1, q1w2_ref[...]) + q1b2_ref[...], 0.0)
            q1 = _dot(h1, q1w3_ref[...])[:, 0:1] + q1b3_ref[0, 0]
            h2 = jnp.maximum(_dot(z, q2w1_ref[...]) + q2b1_ref[...], 0.0)
            h2 = jnp.maximum(_dot(h2, q2w2_ref[...]) + q2b2_ref[...], 0.0)
            q2 = _dot(h2, q2w3_ref[...])[:, 0:1] + q2b3_ref[0, 0]
            q1_ref[...] = q1[0:B, :]
            q2_ref[...] = q2[0:B, :]

    return body


def _tc3(acc2, cnt, bt3, st, ac, g2w2, g2b2, qw, Rb, Bq):
    _, Npad, _ = acc2.shape
    G = g2w2.shape[0]
    H = qw[0].shape[0]
    QIN = qw[0].shape[1]
    grid = (Npad // Rb,)
    body = _make_tc3_body(Bq, G)
    wspecs = [
        pl.BlockSpec((H, QIN), lambda i: (0, 0)),
        pl.BlockSpec((1, H), lambda i: (0, 0)),
        pl.BlockSpec((H, H), lambda i: (0, 0)),
        pl.BlockSpec((1, H), lambda i: (0, 0)),
        pl.BlockSpec((128, H), lambda i: (0, 0)),
        pl.BlockSpec((1, 1), lambda i: (0, 0)),
    ]
    return pl.pallas_call(
        body,
        grid=grid,
        in_specs=[
            pl.BlockSpec((2, Rb, 128), lambda i: (0, i, 0)),
            pl.BlockSpec((2, Rb, 128), lambda i: (0, i, 0)),
            pl.BlockSpec((1, 1, Rb), lambda i: (i, 0, 0)),
            pl.BlockSpec((st.shape[0], st.shape[1]), lambda i: (0, 0)),
            pl.BlockSpec((ac.shape[0], ac.shape[1]), lambda i: (0, 0)),
            pl.BlockSpec((G, G), lambda i: (0, 0)),
            pl.BlockSpec((1, G), lambda i: (0, 0)),
        ] + wspecs + wspecs,
        out_specs=[
            pl.BlockSpec((Bq, 1), lambda i: (0, 0)),
            pl.BlockSpec((Bq, 1), lambda i: (0, 0)),
        ],
        out_shape=[
            jax.ShapeDtypeStruct((Bq, 1), _F32),
            jax.ShapeDtypeStruct((Bq, 1), _F32),
        ],
        scratch_shapes=[pltpu.VMEM((128, 128), _F32)],
        compiler_params=pltpu.CompilerParams(
            dimension_semantics=("arbitrary",)),
    )(acc2, cnt, bt3, st, ac, g2w2, g2b2, *qw)


def _sc_edge(a_nodes, b_nodes, ei, n_nodes, K, packed):
    Npad, Dw = a_nodes.shape
    D = Dw
    Gc = (Dw // 2 if packed else Dw) // 16
    G = D // 16
    Epad = ei.shape[1]
    mesh = plsc.VectorSubcoreMesh(core_axis_name="c", subcore_axis_name="s")
    NC, NS = mesh.num_cores, mesh.num_subcores
    NW = NC * NS
    ALGN = NS * 8
    NROW = min(Npad, ((n_nodes + 1 + ALGN - 1) // ALGN) * ALGN)
    NCHW = Epad // (K * NW)
    NPAIR = NCHW // 2

    out_type = jax.ShapeDtypeStruct((NC, Npad, D), _F32)
    scratch = [
        pltpu.VMEM((K,), jnp.int32),
        pltpu.VMEM((K,), jnp.int32),
        pltpu.VMEM((K,), jnp.int32),
        pltpu.VMEM((K,), jnp.int32),
        pltpu.VMEM((K, Dw), _F32),
        pltpu.VMEM((K, Dw), _F32),
        pltpu.VMEM((K, Dw), _F32),
        pltpu.VMEM((K, Dw), _F32),
        pltpu.VMEM_SHARED((NROW, D), _F32),
        pltpu.SemaphoreType.DMA,
        pltpu.SemaphoreType.DMA,
        pltpu.SemaphoreType.DMA,
        pltpu.SemaphoreType.DMA,
    ]

    def body(a_hbm, b_hbm, ei_hbm, acc_out,
             idxs0, idxs1, idxd0, idxd1, ga0, ga1, gb0, gb1, acc_sh,
             sga0, sga1, sgb0, sgb1):
        idxs = (idxs0, idxs1)
        idxd = (idxd0, idxd1)
        ga = (ga0, ga1)
        gb = (gb0, gb1)
        sga = (sga0, sga1)
        sgb = (sgb0, sgb1)
        cid = lax.axis_index("c")
        sid = lax.axis_index("s")
        wid = sid * NC + cid
        cbase = wid * NCHW
        zv = jnp.zeros((16,), _F32)

        def zrow(i, carry):
            for j in range(G):
                ga0[i, pl.ds(j * 16, 16)] = zv
            return carry

        lax.fori_loop(0, K, zrow, 0)

        RPW = NROW // NS
        nfull, rem = RPW // K, RPW % K
        base = sid * RPW
        for k in range(nfull):
            pltpu.sync_copy(ga0, acc_sh.at[pl.ds(base + k * K, K)])
        if rem:
            pltpu.sync_copy(ga0.at[pl.ds(0, rem)],
                            acc_sh.at[pl.ds(base + nfull * K, rem)])

        TAIL = Npad - NROW

        @pl.when(sid == 0)
        def _():
            t = 0
            while t < TAIL:
                c = min(K, TAIL - t)
                pltpu.sync_copy(ga0.at[pl.ds(0, c)],
                                acc_out.at[cid, pl.ds(NROW + t, c)])
                t += c

        plsc.subcore_barrier()

        def fetch(c, s):
            off = (cbase + c) * K
            pltpu.sync_copy(ei_hbm.at[0, pl.ds(off, K)], idxs[s])
            pltpu.sync_copy(ei_hbm.at[1, pl.ds(off, K)], idxd[s])
            c1 = pltpu.async_copy(a_hbm.at[idxd[s]], ga[s], sga[s])
            c2 = pltpu.async_copy(b_hbm.at[idxs[s]], gb[s], sgb[s])
            return c1, c2

        def consume(s):
            pltpu.make_async_copy(a_hbm.at[idxd[s]], ga[s], sga[s]).wait()
            pltpu.make_async_copy(b_hbm.at[idxs[s]], gb[s], sgb[s]).wait()

            def crow(i, c2):
                for j in range(Gc):
                    sl = pl.ds(j * 16, 16)
                    sb = pl.ds(Dw // 2 + j * 16, 16) if packed else sl
                    ga[s][i, sl] = jnp.maximum(ga[s][i, sl] + gb[s][i, sb],
                                               0.0)
                return c2

            lax.fori_loop(0, K, crow, 0, unroll=2)
            pltpu.sync_copy(ga[s], acc_sh.at[idxd[s]], add=True)

        fetch(0, 0)
        fetch(1, 1)

        def pair(t, carry):
            for s in range(2):
                consume(s)

                @pl.when(t < NPAIR - 1)
                def _():
                    fetch(2 * t + 2 + s, s)

            return carry

        lax.fori_loop(0, NPAIR, pair, 0)
        plsc.subcore_barrier()

        for k in range(nfull):
            r = base + k * K
            pltpu.sync_copy(acc_sh.at[pl.ds(r, K)],
                            acc_out.at[cid, pl.ds(r, K)])
        if rem:
            r = base + nfull * K
            pltpu.sync_copy(acc_sh.at[pl.ds(r, rem)],
                            acc_out.at[cid, pl.ds(r, rem)])

    fn = pl.kernel(body, out_type=out_type, mesh=mesh,
                   scratch_types=scratch)
    return fn(a_nodes, b_nodes, ei)


def _sc_count(ei, n_nodes, Npad, K):
    D = 128
    G = D // 16
    Epad = ei.shape[1]
    mesh = plsc.VectorSubcoreMesh(core_axis_name="c", subcore_axis_name="s")
    NC, NS = mesh.num_cores, mesh.num_subcores
    NW = NC * NS
    ALGN = NS * 8
    NROW = min(Npad, ((n_nodes + 1 + ALGN - 1) // ALGN) * ALGN)
    NCHW = Epad // (K * NW)

    def body(ei_hbm, cnt_out, idxd_v, ones_v, cnt_sh):
        cid = lax.axis_index("c")
        sid = lax.axis_index("s")
        wid = sid * NC + cid
        zv = jnp.zeros((16,), _F32)
        ov = jnp.ones((16,), _F32)

        def zrow(i, carry):
            for j in range(G):
                ones_v[i, pl.ds(j * 16, 16)] = zv
            return carry

        lax.fori_loop(0, K, zrow, 0)

        RPW = NROW // NS
        nfull, rem = RPW // K, RPW % K
        base = sid * RPW
        for k in range(nfull):
            pltpu.sync_copy(ones_v, cnt_sh.at[pl.ds(base + k * K, K)])
        if rem:
            pltpu.sync_copy(ones_v.at[pl.ds(0, rem)],
                            cnt_sh.at[pl.ds(base + nfull * K, rem)])
        TAIL = Npad - NROW

        @pl.when(sid == 0)
        def _():
            t = 0
            while t < TAIL:
                c = min(K, TAIL - t)
                pltpu.sync_copy(ones_v.at[pl.ds(0, c)],
                                cnt_out.at[cid, pl.ds(NROW + t, c)])
                t += c

        def orow(i, carry):
            for j in range(G):
                ones_v[i, pl.ds(j * 16, 16)] = ov
            return carry

        lax.fori_loop(0, K, orow, 0)
        plsc.subcore_barrier()

        def chunk(t, carry):
            off = (wid * NCHW + t) * K
            pltpu.sync_copy(ei_hbm.at[1, pl.ds(off, K)], idxd_v)
            pltpu.sync_copy(ones_v, cnt_sh.at[idxd_v], add=True)
            return carry

        lax.fori_loop(0, NCHW, chunk, 0)
        plsc.subcore_barrier()

        for k in range(nfull):
            r = base + k * K
            pltpu.sync_copy(cnt_sh.at[pl.ds(r, K)],
                            cnt_out.at[cid, pl.ds(r, K)])
        if rem:
            r = base + nfull * K
            pltpu.sync_copy(cnt_sh.at[pl.ds(r, rem)],
                            cnt_out.at[cid, pl.ds(r, rem)])

    fn = pl.kernel(
        body,
        out_type=jax.ShapeDtypeStruct((NC, Npad, D), _F32),
        mesh=mesh,
        scratch_types=[
            pltpu.VMEM((K,), jnp.int32),
            pltpu.VMEM((K, D), _F32),
            pltpu.VMEM_SHARED((NROW, D), _F32),
        ],
    )
    return fn(ei)


def kernel(state, action, x, edge_index, batch, g1w1, g1b1, g1w2, g1b2,
           g2w1, g2b1, g2w2, g2b2, q1w1, q1b1, q1w2, q1b2, q1w3, q1b3,
           q2w1, q2b1, q2w2, q2b2, q2w3, q2b3):
    N, SD = x.shape
    B, AD = action.shape
    E = edge_index.shape[1]
    ND = SD + AD
    GH = g1w2.shape[0]
    G = g2w2.shape[0]
    rep = N // B

    Rb = 512
    ALIGN = 2048
    Npad = ((N + 1 + ALIGN - 1) // ALIGN) * ALIGN
    CHW = _KCH * 64
    Epad = ((E + CHW - 1) // CHW) * CHW

    nf = jnp.concatenate([x, jnp.repeat(action, rep, axis=0)], axis=1)
    nf = jnp.pad(nf, ((0, Npad - N), (0, 0)))
    ei = edge_index
    if Epad != E:
        ei = jnp.concatenate(
            [ei, jnp.full((2, Epad - E), N, edge_index.dtype)], axis=1)

    cnt = _sc_count(ei, N, Npad, _KCH)
    A1, B1 = _tc1(nf, g1w1[:, :ND], g1w1[:, ND:], g1b1.reshape(1, -1), Rb)
    acc1 = _sc_edge(A1, B1, ei, N, K=_KCH, packed=False)
    AB2 = _tc2(acc1, cnt, g1w2, g1b2.reshape(1, -1),
               g2w1[:, :GH], g2w1[:, GH:], g2b1.reshape(1, -1), Rb)
    acc2 = _sc_edge(AB2, AB2, ei, N, K=_KCH, packed=True)

    bt = jnp.pad(batch.astype(jnp.int32), (0, Npad - N),
                 constant_values=min(127, B))
    bt3 = bt.reshape(Npad // Rb, 1, Rb)
    stp = jnp.pad(state, ((0, 128 - B), (0, 0)))
    acp = jnp.pad(action, ((0, 128 - B), (0, 0)))
    q1w3p = jnp.pad(q1w3, ((0, 128 - q1w3.shape[0]), (0, 0)))
    q2w3p = jnp.pad(q2w3, ((0, 128 - q2w3.shape[0]), (0, 0)))
    qw = [q1w1, q1b1.reshape(1, -1), q1w2, q1b2.reshape(1, -1),
          q1w3p, q1b3.reshape(1, -1),
          q2w1, q2b1.reshape(1, -1), q2w2, q2b2.reshape(1, -1),
          q2w3p, q2b3.reshape(1, -1)]
    q1, q2 = _tc3(acc2, cnt, bt3, stp, acp, g2w2, g2b2.reshape(1, -1),
                  qw, Rb, B)
    return (q1, q2)


# speedup vs baseline: 3.7548x; 1.2841x over previous
import functools

import jax
import jax.numpy as jnp
from jax import lax
from jax.experimental import pallas as pl
from jax.experimental.pallas import tpu as pltpu
from jax.experimental.pallas import tpu_sc as plsc

_F32 = jnp.float32
_KCH = 64


def _dot(a, b):
    return lax.dot_general(a, b, (((1,), (1,)), ((), ())),
                           preferred_element_type=_F32,
                           precision=lax.Precision.HIGHEST)


def _tc1_body(nf_ref, wd_ref, ws_ref, b1_ref, a_ref, b_ref):
    nf = nf_ref[...]
    a_ref[...] = _dot(nf, wd_ref[...]) + b1_ref[...]
    b_ref[...] = _dot(nf, ws_ref[...])


def _tc1(nf, wd, ws, b1, Rb):
    Npad, ND = nf.shape
    GH = wd.shape[0]
    grid = (Npad // Rb,)
    return pl.pallas_call(
        _tc1_body,
        grid=grid,
        in_specs=[
            pl.BlockSpec((Rb, ND), lambda i: (i, 0)),
            pl.BlockSpec((GH, ND), lambda i: (0, 0)),
            pl.BlockSpec((GH, ND), lambda i: (0, 0)),
            pl.BlockSpec((1, GH), lambda i: (0, 0)),
        ],
        out_specs=[
            pl.BlockSpec((Rb, GH), lambda i: (i, 0)),
            pl.BlockSpec((Rb, GH), lambda i: (i, 0)),
        ],
        out_shape=[
            jax.ShapeDtypeStruct((Npad, GH), _F32),
            jax.ShapeDtypeStruct((Npad, GH), _F32),
        ],
        compiler_params=pltpu.CompilerParams(
            dimension_semantics=("arbitrary",)),
    )(nf, wd, ws, b1)


def _tc2_body(acc_ref, cnt_ref, w2_ref, b2_ref, wd2_ref, ws2_ref, b21_ref,
              ab_ref):
    S = acc_ref[0] + acc_ref[1]
    cnt = cnt_ref[0, :, 0:1] + cnt_ref[1, :, 0:1]
    mask = (cnt > 0.0).astype(_F32)
    mean = S * (1.0 / jnp.maximum(cnt, 1.0))
    h = jnp.maximum(_dot(mean, w2_ref[...]) + b2_ref[...] * mask, 0.0)
    ab_ref[...] = jnp.concatenate(
        [_dot(h, wd2_ref[...]) + b21_ref[...], _dot(h, ws2_ref[...])],
        axis=1)


def _tc2(acc, cnt, w2, b2, wd2, ws2, b21, Rb):
    _, Npad, GH = acc.shape
    G = wd2.shape[0]
    grid = (Npad // Rb,)
    return pl.pallas_call(
        _tc2_body,
        grid=grid,
        in_specs=[
            pl.BlockSpec((2, Rb, GH), lambda i: (0, i, 0)),
            pl.BlockSpec((2, Rb, 128), lambda i: (0, i, 0)),
            pl.BlockSpec((GH, GH), lambda i: (0, 0)),
            pl.BlockSpec((1, GH), lambda i: (0, 0)),
            pl.BlockSpec((G, GH), lambda i: (0, 0)),
            pl.BlockSpec((G, GH), lambda i: (0, 0)),
            pl.BlockSpec((1, G), lambda i: (0, 0)),
        ],
        out_specs=pl.BlockSpec((Rb, 2 * G), lambda i: (i, 0)),
        out_shape=jax.ShapeDtypeStruct((Npad, 2 * G), _F32),
        compiler_params=pltpu.CompilerParams(
            dimension_semantics=("arbitrary",)),
    )(acc, cnt, w2, b2, wd2, ws2, b21)


def _make_tc3_body(B, G):
    def body(acc2_ref, cnt_ref, bt_ref, st_ref, ac_ref, g2w2_ref, g2b2_ref,
             q1w1_ref, q1b1_ref, q1w2_ref, q1b2_ref, q1w3_ref, q1b3_ref,
             q2w1_ref, q2b1_ref, q2w2_ref, q2b2_ref, q2w3_ref, q2b3_ref,
             q1_ref, q2_ref, accP):
        i = pl.program_id(0)
        nsteps = pl.num_programs(0)

        @pl.when(i == 0)
        def _():
            accP[...] = jnp.zeros_like(accP)

        S2 = (acc2_ref[0] + acc2_ref[1])[:, 0:G]
        cnt = cnt_ref[0, :, 0:1] + cnt_ref[1, :, 0:1]
        mask = (cnt > 0.0).astype(_F32)
        P = S2 * (1.0 / jnp.maximum(cnt, 1.0))
        Rb = P.shape[0]
        ones = jnp.ones((Rb, 1), _F32)
        zrest = jnp.zeros((Rb, 128 - G - 2), _F32)
        P2 = jnp.concatenate([P, mask, ones, zrest], axis=1)
        bt = bt_ref[0]
        iot = lax.broadcasted_iota(jnp.int32, (128, Rb), 0)
        ohT = (iot == bt).astype(_F32)
        accP[...] += lax.dot_general(ohT, P2, (((1,), (0,)), ((), ())),
                                     preferred_element_type=_F32,
                                     precision=lax.Precision.HIGHEST)

        @pl.when(i == nsteps - 1)
        def _():
            A = accP[...]
            geS = A[:, 0:G]
            gm = A[:, G:G + 1]
            gn = A[:, G + 1:G + 2]
            ge = (_dot(geS, g2w2_ref[...]) + g2b2_ref[...] * gm) * (
                1.0 / jnp.maximum(gn, 1.0))
            z = jnp.concatenate([st_ref[...], ac_ref[...], ge], axis=1)
            h1 = jnp.maximum(_dot(z, q1w1_ref[...]) + q1b1_ref[...], 0.0)
            h1 = jnp.maximum(_dot(h1, q1w2_ref[...]) + q1b2_ref[...], 0.0)
            q1 = _dot(h1, q1w3_ref[...])[:, 0:1] + q1b3_ref[0, 0]
            h2 = jnp.maximum(_dot(z, q2w1_ref[...]) + q2b1_ref[...], 0.0)
            h2 = jnp.maximum(_dot(h2, q2w2_ref[...]) + q2b2_ref[...], 0.0)
            q2 = _dot(h2, q2w3_ref[...])[:, 0:1] + q2b3_ref[0, 0]
            q1_ref[...] = q1[0:B, :]
            q2_ref[...] = q2[0:B, :]

    return body


def _tc3(acc2, cnt, bt3, st, ac, g2w2, g2b2, qw, Rb, Bq):
    _, Npad, _ = acc2.shape
    G = g2w2.shape[0]
    H = qw[0].shape[0]
    QIN = qw[0].shape[1]
    grid = (Npad // Rb,)
    body = _make_tc3_body(Bq, G)
    wspecs = [
        pl.BlockSpec((H, QIN), lambda i: (0, 0)),
        pl.BlockSpec((1, H), lambda i: (0, 0)),
        pl.BlockSpec((H, H), lambda i: (0, 0)),
        pl.BlockSpec((1, H), lambda i: (0, 0)),
        pl.BlockSpec((128, H), lambda i: (0, 0)),
        pl.BlockSpec((1, 1), lambda i: (0, 0)),
    ]
    return pl.pallas_call(
        body,
        grid=grid,
        in_specs=[
            pl.BlockSpec((2, Rb, 128), lambda i: (0, i, 0)),
            pl.BlockSpec((2, Rb, 128), lambda i: (0, i, 0)),
            pl.BlockSpec((1, 1, Rb), lambda i: (i, 0, 0)),
            pl.BlockSpec((st.shape[0], st.shape[1]), lambda i: (0, 0)),
            pl.BlockSpec((ac.shape[0], ac.shape[1]), lambda i: (0, 0)),
            pl.BlockSpec((G, G), lambda i: (0, 0)),
            pl.BlockSpec((1, G), lambda i: (0, 0)),
        ] + wspecs + wspecs,
        out_specs=[
            pl.BlockSpec((Bq, 1), lambda i: (0, 0)),
            pl.BlockSpec((Bq, 1), lambda i: (0, 0)),
        ],
        out_shape=[
            jax.ShapeDtypeStruct((Bq, 1), _F32),
            jax.ShapeDtypeStruct((Bq, 1), _F32),
        ],
        scratch_shapes=[pltpu.VMEM((128, 128), _F32)],
        compiler_params=pltpu.CompilerParams(
            dimension_semantics=("arbitrary",)),
    )(acc2, cnt, bt3, st, ac, g2w2, g2b2, *qw)


def _sc_edge(a_nodes, b_nodes, ei3, n_nodes, K, packed):
    Npad, Dw = a_nodes.shape
    D = Dw
    Gc = (Dw // 2 if packed else Dw) // 16
    G = D // 16
    NCH = ei3.shape[0]
    mesh = plsc.VectorSubcoreMesh(core_axis_name="c", subcore_axis_name="s")
    NC, NS = mesh.num_cores, mesh.num_subcores
    NW = NC * NS
    ALGN = NS * 8
    NROW = min(Npad, ((n_nodes + 1 + ALGN - 1) // ALGN) * ALGN)
    NCHW = NCH // NW
    NPAIR = NCHW // 2

    out_type = jax.ShapeDtypeStruct((NC, Npad, D), _F32)
    scratch = [
        pltpu.VMEM((2, K), jnp.int32),
        pltpu.VMEM((2, K), jnp.int32),
        pltpu.VMEM((K, Dw), _F32),
        pltpu.VMEM((K, Dw), _F32),
        pltpu.VMEM((K, Dw), _F32),
        pltpu.VMEM((K, Dw), _F32),
        pltpu.VMEM_SHARED((NROW, D), _F32),
        pltpu.SemaphoreType.DMA,
        pltpu.SemaphoreType.DMA,
        pltpu.SemaphoreType.DMA,
        pltpu.SemaphoreType.DMA,
        pltpu.SemaphoreType.DMA,
        pltpu.SemaphoreType.DMA,
    ]

    def body(a_hbm, b_hbm, ei_hbm, acc_out,
             idx0, idx1, ga0, ga1, gb0, gb1, acc_sh,
             sga0, sga1, sgb0, sgb1, si0, si1):
        idx = (idx0, idx1)
        ga = (ga0, ga1)
        gb = (gb0, gb1)
        sga = (sga0, sga1)
        sgb = (sgb0, sgb1)
        si = (si0, si1)
        cid = lax.axis_index("c")
        sid = lax.axis_index("s")
        wid = sid * NC + cid
        cbase = wid * NCHW
        zv = jnp.zeros((16,), _F32)

        def zrow(i, carry):
            for j in range(G):
                ga0[i, pl.ds(j * 16, 16)] = zv
            return carry

        lax.fori_loop(0, K, zrow, 0)

        RPW = NROW // NS
        nfull, rem = RPW // K, RPW % K
        base = sid * RPW
        for k in range(nfull):
            pltpu.sync_copy(ga0, acc_sh.at[pl.ds(base + k * K, K)])
        if rem:
            pltpu.sync_copy(ga0.at[pl.ds(0, rem)],
                            acc_sh.at[pl.ds(base + nfull * K, rem)])

        TAIL = Npad - NROW

        @pl.when(sid == 0)
        def _():
            t = 0
            while t < TAIL:
                c = min(K, TAIL - t)
                pltpu.sync_copy(ga0.at[pl.ds(0, c)],
                                acc_out.at[cid, pl.ds(NROW + t, c)])
                t += c

        plsc.subcore_barrier()

        def gathers(c, s):
            pltpu.async_copy(a_hbm.at[idx[s].at[1]], ga[s], sga[s])
            pltpu.async_copy(b_hbm.at[idx[s].at[0]], gb[s], sgb[s])

        def step(t, s):
            c = cbase + 2 * t + s
            pltpu.make_async_copy(a_hbm.at[idx[s].at[1]], ga[s],
                                  sga[s]).wait()
            pltpu.make_async_copy(b_hbm.at[idx[s].at[0]], gb[s],
                                  sgb[s]).wait()

            @pl.when(t < NPAIR - 1)
            def _():
                pltpu.async_copy(ei_hbm.at[c + 2], idx[s], si[s])

            def crow(i, c2):
                for j in range(Gc):
                    sl = pl.ds(j * 16, 16)
                    sb = pl.ds(Dw // 2 + j * 16, 16) if packed else sl
                    ga[s][i, sl] = jnp.maximum(ga[s][i, sl] + gb[s][i, sb],
                                               0.0)
                return c2

            lax.fori_loop(0, K, crow, 0, unroll=2)
            pltpu.sync_copy(ga[s], acc_sh.at[idx[s].at[1]], add=True)

            @pl.when(t < NPAIR - 1)
            def _():
                pltpu.make_async_copy(ei_hbm.at[c + 2], idx[s], si[s]).wait()
                gathers(c + 2, s)

        pltpu.sync_copy(ei_hbm.at[cbase], idx0)
        pltpu.sync_copy(ei_hbm.at[cbase + 1], idx1)
        gathers(cbase, 0)
        gathers(cbase + 1, 1)

        def pair(t, carry):
            step(t, 0)
            step(t, 1)
            return carry

        lax.fori_loop(0, NPAIR, pair, 0)
        plsc.subcore_barrier()

        for k in range(nfull):
            r = base + k * K
            pltpu.sync_copy(acc_sh.at[pl.ds(r, K)],
                            acc_out.at[cid, pl.ds(r, K)])
        if rem:
            r = base + nfull * K
            pltpu.sync_copy(acc_sh.at[pl.ds(r, rem)],
                            acc_out.at[cid, pl.ds(r, rem)])

    fn = pl.kernel(body, out_type=out_type, mesh=mesh,
                   scratch_types=scratch)
    return fn(a_nodes, b_nodes, ei3)


def _sc_count(ei3, n_nodes, Npad, K):
    D = 128
    G = D // 16
    NCH = ei3.shape[0]
    mesh = plsc.VectorSubcoreMesh(core_axis_name="c", subcore_axis_name="s")
    NC, NS = mesh.num_cores, mesh.num_subcores
    NW = NC * NS
    ALGN = NS * 8
    NROW = min(Npad, ((n_nodes + 1 + ALGN - 1) // ALGN) * ALGN)
    NCHW = NCH // NW
    NPAIR = NCHW // 2

    def body(ei_hbm, cnt_out, idx0, idx1, ones_v, cnt_sh, si0, si1):
        idx = (idx0, idx1)
        si = (si0, si1)
        cid = lax.axis_index("c")
        sid = lax.axis_index("s")
        wid = sid * NC + cid
        cbase = wid * NCHW
        zv = jnp.zeros((16,), _F32)
        ov = jnp.ones((16,), _F32)

        def zrow(i, carry):
            for j in range(G):
                ones_v[i, pl.ds(j * 16, 16)] = zv
            return carry

        lax.fori_loop(0, K, zrow, 0)

        RPW = NROW // NS
        nfull, rem = RPW // K, RPW % K
        base = sid * RPW
        for k in range(nfull):
            pltpu.sync_copy(ones_v, cnt_sh.at[pl.ds(base + k * K, K)])
        if rem:
            pltpu.sync_copy(ones_v.at[pl.ds(0, rem)],
                            cnt_sh.at[pl.ds(base + nfull * K, rem)])
        TAIL = Npad - NROW

        @pl.when(sid == 0)
        def _():
            t = 0
            while t < TAIL:
                c = min(K, TAIL - t)
                pltpu.sync_copy(ones_v.at[pl.ds(0, c)],
                                cnt_out.at[cid, pl.ds(NROW + t, c)])
                t += c

        def orow(i, carry):
            for j in range(G):
                ones_v[i, pl.ds(j * 16, 16)] = ov
            return carry

        lax.fori_loop(0, K, orow, 0)
        plsc.subcore_barrier()

        def step(t, s):
            c = cbase + 2 * t + s

            @pl.when(t > 0)
            def _():
                pltpu.make_async_copy(ei_hbm.at[c], idx[s], si[s]).wait()

            @pl.when(t < NPAIR - 1)
            def _():
                pltpu.async_copy(ei_hbm.at[c + 2], idx[s], si[s])

            pltpu.sync_copy(ones_v, cnt_sh.at[idx[s].at[1]], add=True)

        pltpu.sync_copy(ei_hbm.at[cbase], idx0)
        pltpu.sync_copy(ei_hbm.at[cbase + 1], idx1)

        def pair(t, carry):
            step(t, 0)
            step(t, 1)
            return carry

        lax.fori_loop(0, NPAIR, pair, 0)
        plsc.subcore_barrier()

        for k in range(nfull):
            r = base + k * K
            pltpu.sync_copy(cnt_sh.at[pl.ds(r, K)],
                            cnt_out.at[cid, pl.ds(r, K)])
        if rem:
            r = base + nfull * K
            pltpu.sync_copy(cnt_sh.at[pl.ds(r, rem)],
                            cnt_out.at[cid, pl.ds(r, rem)])

    fn = pl.kernel(
        body,
        out_type=jax.ShapeDtypeStruct((NC, Npad, D), _F32),
        mesh=mesh,
        scratch_types=[
            pltpu.VMEM((2, K), jnp.int32),
            pltpu.VMEM((2, K), jnp.int32),
            pltpu.VMEM((K, D), _F32),
            pltpu.VMEM_SHARED((NROW, D), _F32),
            pltpu.SemaphoreType.DMA,
            pltpu.SemaphoreType.DMA,
        ],
    )
    return fn(ei3)


def kernel(state, action, x, edge_index, batch, g1w1, g1b1, g1w2, g1b2,
           g2w1, g2b1, g2w2, g2b2, q1w1, q1b1, q1w2, q1b2, q1w3, q1b3,
           q2w1, q2b1, q2w2, q2b2, q2w3, q2b3):
    N, SD = x.shape
    B, AD = action.shape
    E = edge_index.shape[1]
    ND = SD + AD
    GH = g1w2.shape[0]
    G = g2w2.shape[0]
    rep = N // B

    Rb = 512
    ALIGN = 2048
    Npad = ((N + 1 + ALIGN - 1) // ALIGN) * ALIGN
    CHW = _KCH * 64
    Epad = ((E + CHW - 1) // CHW) * CHW

    nf = jnp.concatenate([x, jnp.repeat(action, rep, axis=0)], axis=1)
    nf = jnp.pad(nf, ((0, Npad - N), (0, 0)))
    ei = edge_index
    if Epad != E:
        ei = jnp.concatenate(
            [ei, jnp.full((2, Epad - E), N, edge_index.dtype)], axis=1)
    ei3 = ei.astype(jnp.int32).reshape(2, Epad // _KCH, _KCH).transpose(1, 0, 2)

    cnt = _sc_count(ei3, N, Npad, _KCH)
    A1, B1 = _tc1(nf, g1w1[:, :ND], g1w1[:, ND:], g1b1.reshape(1, -1), Rb)
    acc1 = _sc_edge(A1, B1, ei3, N, K=_KCH, packed=False)
    AB2 = _tc2(acc1, cnt, g1w2, g1b2.reshape(1, -1),
               g2w1[:, :GH], g2w1[:, GH:], g2b1.reshape(1, -1), Rb)
    acc2 = _sc_edge(AB2, AB2, ei3, N, K=_KCH, packed=True)

    bt = jnp.pad(batch.astype(jnp.int32), (0, Npad - N),
                 constant_values=min(127, B))
    bt3 = bt.reshape(Npad // Rb, 1, Rb)
    stp = jnp.pad(state, ((0, 128 - B), (0, 0)))
    acp = jnp.pad(action, ((0, 128 - B), (0, 0)))
    q1w3p = jnp.pad(q1w3, ((0, 128 - q1w3.shape[0]), (0, 0)))
    q2w3p = jnp.pad(q2w3, ((0, 128 - q2w3.shape[0]), (0, 0)))
    qw = [q1w1, q1b1.reshape(1, -1), q1w2, q1b2.reshape(1, -1),
          q1w3p, q1b3.reshape(1, -1),
          q2w1, q2b1.reshape(1, -1), q2w2, q2b2.reshape(1, -1),
          q2w3p, q2b3.reshape(1, -1)]
    q1, q2 = _tc3(acc2, cnt, bt3, stp, acp, g2w2, g2b2.reshape(1, -1),
                  qw, Rb, B)
    return (q1, q2)


# speedup vs baseline: 3.8365x; 1.0217x over previous
import functools

import jax
import jax.numpy as jnp
from jax import lax
from jax.experimental import pallas as pl
from jax.experimental.pallas import tpu as pltpu
from jax.experimental.pallas import tpu_sc as plsc

_F32 = jnp.float32
_KCH = 80


def _dot(a, b):
    return lax.dot_general(a, b, (((1,), (1,)), ((), ())),
                           preferred_element_type=_F32,
                           precision=lax.Precision.HIGHEST)


def _tc1_body(nf_ref, wd_ref, ws_ref, b1_ref, a_ref, b_ref):
    nf = nf_ref[...]
    a_ref[...] = _dot(nf, wd_ref[...]) + b1_ref[...]
    b_ref[...] = _dot(nf, ws_ref[...])


def _tc1(nf, wd, ws, b1, Rb):
    Npad, ND = nf.shape
    GH = wd.shape[0]
    grid = (Npad // Rb,)
    return pl.pallas_call(
        _tc1_body,
        grid=grid,
        in_specs=[
            pl.BlockSpec((Rb, ND), lambda i: (i, 0)),
            pl.BlockSpec((GH, ND), lambda i: (0, 0)),
            pl.BlockSpec((GH, ND), lambda i: (0, 0)),
            pl.BlockSpec((1, GH), lambda i: (0, 0)),
        ],
        out_specs=[
            pl.BlockSpec((Rb, GH), lambda i: (i, 0)),
            pl.BlockSpec((Rb, GH), lambda i: (i, 0)),
        ],
        out_shape=[
            jax.ShapeDtypeStruct((Npad, GH), _F32),
            jax.ShapeDtypeStruct((Npad, GH), _F32),
        ],
        compiler_params=pltpu.CompilerParams(
            dimension_semantics=("arbitrary",)),
    )(nf, wd, ws, b1)


def _tc2_body(acc_ref, cnt_ref, w2_ref, b2_ref, wd2_ref, ws2_ref, b21_ref,
              ab_ref):
    S = acc_ref[0] + acc_ref[1]
    cnt = cnt_ref[0, :, 0:1] + cnt_ref[1, :, 0:1]
    mask = (cnt > 0.0).astype(_F32)
    mean = S * (1.0 / jnp.maximum(cnt, 1.0))
    h = jnp.maximum(_dot(mean, w2_ref[...]) + b2_ref[...] * mask, 0.0)
    ab_ref[...] = jnp.concatenate(
        [_dot(h, wd2_ref[...]) + b21_ref[...], _dot(h, ws2_ref[...])],
        axis=1)


def _tc2(acc, cnt, w2, b2, wd2, ws2, b21, Rb):
    _, Npad, GH = acc.shape
    G = wd2.shape[0]
    grid = (Npad // Rb,)
    return pl.pallas_call(
        _tc2_body,
        grid=grid,
        in_specs=[
            pl.BlockSpec((2, Rb, GH), lambda i: (0, i, 0)),
            pl.BlockSpec((2, Rb, 128), lambda i: (0, i, 0)),
            pl.BlockSpec((GH, GH), lambda i: (0, 0)),
            pl.BlockSpec((1, GH), lambda i: (0, 0)),
            pl.BlockSpec((G, GH), lambda i: (0, 0)),
            pl.BlockSpec((G, GH), lambda i: (0, 0)),
            pl.BlockSpec((1, G), lambda i: (0, 0)),
        ],
        out_specs=pl.BlockSpec((Rb, 2 * G), lambda i: (i, 0)),
        out_shape=jax.ShapeDtypeStruct((Npad, 2 * G), _F32),
        compiler_params=pltpu.CompilerParams(
            dimension_semantics=("arbitrary",)),
    )(acc, cnt, w2, b2, wd2, ws2, b21)


def _make_tc3_body(B, G):
    def body(acc2_ref, cnt_ref, bt_ref, st_ref, ac_ref, g2w2_ref, g2b2_ref,
             q1w1_ref, q1b1_ref, q1w2_ref, q1b2_ref, q1w3_ref, q1b3_ref,
             q2w1_ref, q2b1_ref, q2w2_ref, q2b2_ref, q2w3_ref, q2b3_ref,
             q1_ref, q2_ref, accP):
        i = pl.program_id(0)
        nsteps = pl.num_programs(0)

        @pl.when(i == 0)
        def _():
            accP[...] = jnp.zeros_like(accP)

        S2 = (acc2_ref[0] + acc2_ref[1])[:, 0:G]
        cnt = cnt_ref[0, :, 0:1] + cnt_ref[1, :, 0:1]
        mask = (cnt > 0.0).astype(_F32)
        P = S2 * (1.0 / jnp.maximum(cnt, 1.0))
        Rb = P.shape[0]
        ones = jnp.ones((Rb, 1), _F32)
        zrest = jnp.zeros((Rb, 128 - G - 2), _F32)
        P2 = jnp.concatenate([P, mask, ones, zrest], axis=1)
        bt = bt_ref[0]
        iot = lax.broadcasted_iota(jnp.int32, (128, Rb), 0)
        ohT = (iot == bt).astype(_F32)
        accP[...] += lax.dot_general(ohT, P2, (((1,), (0,)), ((), ())),
                                     preferred_element_type=_F32,
                                     precision=lax.Precision.HIGHEST)

        @pl.when(i == nsteps - 1)
        def _():
            A = accP[...]
            geS = A[:, 0:G]
            gm = A[:, G:G + 1]
            gn = A[:, G + 1:G + 2]
            ge = (_dot(geS, g2w2_ref[...]) + g2b2_ref[...] * gm) * (
                1.0 / jnp.maximum(gn, 1.0))
            z = jnp.concatenate([st_ref[...], ac_ref[...], ge], axis=1)
            h1 = jnp.maximum(_dot(z, q1w1_ref[...]) + q1b1_ref[...], 0.0)
            h1 = jnp.maximum(_dot(h1, q1w2_ref[...]) + q1b2_ref[...], 0.0)
            q1 = _dot(h1, q1w3_ref[...])[:, 0:1] + q1b3_ref[0, 0]
            h2 = jnp.maximum(_dot(z, q2w1_ref[...]) + q2b1_ref[...], 0.0)
            h2 = jnp.maximum(_dot(h2, q2w2_ref[...]) + q2b2_ref[...], 0.0)
            q2 = _dot(h2, q2w3_ref[...])[:, 0:1] + q2b3_ref[0, 0]
            q1_ref[...] = q1[0:B, :]
            q2_ref[...] = q2[0:B, :]

    return body


def _tc3(acc2, cnt, bt3, st, ac, g2w2, g2b2, qw, Rb, Bq):
    _, Npad, _ = acc2.shape
    G = g2w2.shape[0]
    H = qw[0].shape[0]
    QIN = qw[0].shape[1]
    grid = (Npad // Rb,)
    body = _make_tc3_body(Bq, G)
    wspecs = [
        pl.BlockSpec((H, QIN), lambda i: (0, 0)),
        pl.BlockSpec((1, H), lambda i: (0, 0)),
        pl.BlockSpec((H, H), lambda i: (0, 0)),
        pl.BlockSpec((1, H), lambda i: (0, 0)),
        pl.BlockSpec((128, H), lambda i: (0, 0)),
        pl.BlockSpec((1, 1), lambda i: (0, 0)),
    ]
    return pl.pallas_call(
        body,
        grid=grid,
        in_specs=[
            pl.BlockSpec((2, Rb, 128), lambda i: (0, i, 0)),
            pl.BlockSpec((2, Rb, 128), lambda i: (0, i, 0)),
            pl.BlockSpec((1, 1, Rb), lambda i: (i, 0, 0)),
            pl.BlockSpec((st.shape[0], st.shape[1]), lambda i: (0, 0)),
            pl.BlockSpec((ac.shape[0], ac.shape[1]), lambda i: (0, 0)),
            pl.BlockSpec((G, G), lambda i: (0, 0)),
            pl.BlockSpec((1, G), lambda i: (0, 0)),
        ] + wspecs + wspecs,
        out_specs=[
            pl.BlockSpec((Bq, 1), lambda i: (0, 0)),
            pl.BlockSpec((Bq, 1), lambda i: (0, 0)),
        ],
        out_shape=[
            jax.ShapeDtypeStruct((Bq, 1), _F32),
            jax.ShapeDtypeStruct((Bq, 1), _F32),
        ],
        scratch_shapes=[pltpu.VMEM((128, 128), _F32)],
        compiler_params=pltpu.CompilerParams(
            dimension_semantics=("arbitrary",)),
    )(acc2, cnt, bt3, st, ac, g2w2, g2b2, *qw)


def _sc_edge(a_nodes, b_nodes, ei3, n_nodes, K, packed):
    Npad, Dw = a_nodes.shape
    D = Dw
    Gc = (Dw // 2 if packed else Dw) // 16
    G = D // 16
    NCH = ei3.shape[0]
    mesh = plsc.VectorSubcoreMesh(core_axis_name="c", subcore_axis_name="s")
    NC, NS = mesh.num_cores, mesh.num_subcores
    NW = NC * NS
    ALGN = NS * 8
    NROW = min(Npad, ((n_nodes + 1 + ALGN - 1) // ALGN) * ALGN)
    NCHW = NCH // NW
    NPAIR = NCHW // 2

    out_type = jax.ShapeDtypeStruct((NC, Npad, D), _F32)
    scratch = [
        pltpu.VMEM((2, K), jnp.int32),
        pltpu.VMEM((2, K), jnp.int32),
        pltpu.VMEM((K, Dw), _F32),
        pltpu.VMEM((K, Dw), _F32),
        pltpu.VMEM((K, Dw), _F32),
        pltpu.VMEM((K, Dw), _F32),
        pltpu.VMEM_SHARED((NROW, D), _F32),
        pltpu.SemaphoreType.DMA,
        pltpu.SemaphoreType.DMA,
        pltpu.SemaphoreType.DMA,
        pltpu.SemaphoreType.DMA,
        pltpu.SemaphoreType.DMA,
        pltpu.SemaphoreType.DMA,
    ]

    def body(a_hbm, b_hbm, ei_hbm, acc_out,
             idx0, idx1, ga0, ga1, gb0, gb1, acc_sh,
             sga0, sga1, sgb0, sgb1, si0, si1):
        idx = (idx0, idx1)
        ga = (ga0, ga1)
        gb = (gb0, gb1)
        sga = (sga0, sga1)
        sgb = (sgb0, sgb1)
        si = (si0, si1)
        cid = lax.axis_index("c")
        sid = lax.axis_index("s")
        wid = sid * NC + cid
        cbase = wid * NCHW
        zv = jnp.zeros((16,), _F32)

        def zrow(i, carry):
            for j in range(G):
                ga0[i, pl.ds(j * 16, 16)] = zv
            return carry

        lax.fori_loop(0, K, zrow, 0)

        RPW = NROW // NS
        nfull, rem = RPW // K, RPW % K
        base = sid * RPW
        for k in range(nfull):
            pltpu.sync_copy(ga0, acc_sh.at[pl.ds(base + k * K, K)])
        if rem:
            pltpu.sync_copy(ga0.at[pl.ds(0, rem)],
                            acc_sh.at[pl.ds(base + nfull * K, rem)])

        TAIL = Npad - NROW

        @pl.when(sid == 0)
        def _():
            t = 0
            while t < TAIL:
                c = min(K, TAIL - t)
                pltpu.sync_copy(ga0.at[pl.ds(0, c)],
                                acc_out.at[cid, pl.ds(NROW + t, c)])
                t += c

        plsc.subcore_barrier()

        def gathers(c, s):
            pltpu.async_copy(a_hbm.at[idx[s].at[1]], ga[s], sga[s])
            pltpu.async_copy(b_hbm.at[idx[s].at[0]], gb[s], sgb[s])

        def step(t, s):
            c = cbase + 2 * t + s
            pltpu.make_async_copy(a_hbm.at[idx[s].at[1]], ga[s],
                                  sga[s]).wait()
            pltpu.make_async_copy(b_hbm.at[idx[s].at[0]], gb[s],
                                  sgb[s]).wait()

            @pl.when(t < NPAIR - 1)
            def _():
                pltpu.async_copy(ei_hbm.at[c + 2], idx[s], si[s])

            def crow(i, c2):
                for j in range(Gc):
                    sl = pl.ds(j * 16, 16)
                    sb = pl.ds(Dw // 2 + j * 16, 16) if packed else sl
                    ga[s][i, sl] = jnp.maximum(ga[s][i, sl] + gb[s][i, sb],
                                               0.0)
                return c2

            lax.fori_loop(0, K, crow, 0, unroll=2)
            pltpu.sync_copy(ga[s], acc_sh.at[idx[s].at[1]], add=True)

            @pl.when(t < NPAIR - 1)
            def _():
                pltpu.make_async_copy(ei_hbm.at[c + 2], idx[s], si[s]).wait()
                gathers(c + 2, s)

        pltpu.sync_copy(ei_hbm.at[cbase], idx0)
        pltpu.sync_copy(ei_hbm.at[cbase + 1], idx1)
        gathers(cbase, 0)
        gathers(cbase + 1, 1)

        def pair(t, carry):
            step(t, 0)
            step(t, 1)
            return carry

        lax.fori_loop(0, NPAIR, pair, 0)
        plsc.subcore_barrier()

        for k in range(nfull):
            r = base + k * K
            pltpu.sync_copy(acc_sh.at[pl.ds(r, K)],
                            acc_out.at[cid, pl.ds(r, K)])
        if rem:
            r = base + nfull * K
            pltpu.sync_copy(acc_sh.at[pl.ds(r, rem)],
                            acc_out.at[cid, pl.ds(r, rem)])

    fn = pl.kernel(body, out_type=out_type, mesh=mesh,
                   scratch_types=scratch)
    return fn(a_nodes, b_nodes, ei3)


def _sc_count(ei3, n_nodes, Npad, K):
    D = 128
    G = D // 16
    NCH = ei3.shape[0]
    mesh = plsc.VectorSubcoreMesh(core_axis_name="c", subcore_axis_name="s")
    NC, NS = mesh.num_cores, mesh.num_subcores
    NW = NC * NS
    ALGN = NS * 8
    NROW = min(Npad, ((n_nodes + 1 + ALGN - 1) // ALGN) * ALGN)
    NCHW = NCH // NW
    NPAIR = NCHW // 2

    def body(ei_hbm, cnt_out, idx0, idx1, ones_v, cnt_sh, si0, si1):
        idx = (idx0, idx1)
        si = (si0, si1)
        cid = lax.axis_index("c")
        sid = lax.axis_index("s")
        wid = sid * NC + cid
        cbase = wid * NCHW
        zv = jnp.zeros((16,), _F32)
        ov = jnp.ones((16,), _F32)

        def zrow(i, carry):
            for j in range(G):
                ones_v[i, pl.ds(j * 16, 16)] = zv
            return carry

        lax.fori_loop(0, K, zrow, 0)

        RPW = NROW // NS
        nfull, rem = RPW // K, RPW % K
        base = sid * RPW
        for k in range(nfull):
            pltpu.sync_copy(ones_v, cnt_sh.at[pl.ds(base + k * K, K)])
        if rem:
            pltpu.sync_copy(ones_v.at[pl.ds(0, rem)],
                            cnt_sh.at[pl.ds(base + nfull * K, rem)])
        TAIL = Npad - NROW

        @pl.when(sid == 0)
        def _():
            t = 0
            while t < TAIL:
                c = min(K, TAIL - t)
                pltpu.sync_copy(ones_v.at[pl.ds(0, c)],
                                cnt_out.at[cid, pl.ds(NROW + t, c)])
                t += c

        def orow(i, carry):
            for j in range(G):
                ones_v[i, pl.ds(j * 16, 16)] = ov
            return carry

        lax.fori_loop(0, K, orow, 0)
        plsc.subcore_barrier()

        def step(t, s):
            c = cbase + 2 * t + s

            @pl.when(t > 0)
            def _():
                pltpu.make_async_copy(ei_hbm.at[c], idx[s], si[s]).wait()

            @pl.when(t < NPAIR - 1)
            def _():
                pltpu.async_copy(ei_hbm.at[c + 2], idx[s], si[s])

            pltpu.sync_copy(ones_v, cnt_sh.at[idx[s].at[1]], add=True)

        pltpu.sync_copy(ei_hbm.at[cbase], idx0)
        pltpu.sync_copy(ei_hbm.at[cbase + 1], idx1)

        def pair(t, carry):
            step(t, 0)
            step(t, 1)
            return carry

        lax.fori_loop(0, NPAIR, pair, 0)
        plsc.subcore_barrier()

        for k in range(nfull):
            r = base + k * K
            pltpu.sync_copy(cnt_sh.at[pl.ds(r, K)],
                            cnt_out.at[cid, pl.ds(r, K)])
        if rem:
            r = base + nfull * K
            pltpu.sync_copy(cnt_sh.at[pl.ds(r, rem)],
                            cnt_out.at[cid, pl.ds(r, rem)])

    fn = pl.kernel(
        body,
        out_type=jax.ShapeDtypeStruct((NC, Npad, D), _F32),
        mesh=mesh,
        scratch_types=[
            pltpu.VMEM((2, K), jnp.int32),
            pltpu.VMEM((2, K), jnp.int32),
            pltpu.VMEM((K, D), _F32),
            pltpu.VMEM_SHARED((NROW, D), _F32),
            pltpu.SemaphoreType.DMA,
            pltpu.SemaphoreType.DMA,
        ],
    )
    return fn(ei3)


def kernel(state, action, x, edge_index, batch, g1w1, g1b1, g1w2, g1b2,
           g2w1, g2b1, g2w2, g2b2, q1w1, q1b1, q1w2, q1b2, q1w3, q1b3,
           q2w1, q2b1, q2w2, q2b2, q2w3, q2b3):
    N, SD = x.shape
    B, AD = action.shape
    E = edge_index.shape[1]
    ND = SD + AD
    GH = g1w2.shape[0]
    G = g2w2.shape[0]
    rep = N // B

    Rb = 512
    ALIGN = 2048
    Npad = ((N + 1 + ALIGN - 1) // ALIGN) * ALIGN
    CHW = _KCH * 64
    Epad = ((E + CHW - 1) // CHW) * CHW

    nf = jnp.concatenate([x, jnp.repeat(action, rep, axis=0)], axis=1)
    nf = jnp.pad(nf, ((0, Npad - N), (0, 0)))
    ei = edge_index
    if Epad != E:
        ei = jnp.concatenate(
            [ei, jnp.full((2, Epad - E), N, edge_index.dtype)], axis=1)
    ei3 = ei.astype(jnp.int32).reshape(2, Epad // _KCH, _KCH).transpose(1, 0, 2)

    cnt = _sc_count(ei3, N, Npad, _KCH)
    A1, B1 = _tc1(nf, g1w1[:, :ND], g1w1[:, ND:], g1b1.reshape(1, -1), Rb)
    acc1 = _sc_edge(A1, B1, ei3, N, K=_KCH, packed=False)
    AB2 = _tc2(acc1, cnt, g1w2, g1b2.reshape(1, -1),
               g2w1[:, :GH], g2w1[:, GH:], g2b1.reshape(1, -1), Rb)
    acc2 = _sc_edge(AB2, AB2, ei3, N, K=_KCH, packed=True)

    bt = jnp.pad(batch.astype(jnp.int32), (0, Npad - N),
                 constant_values=min(127, B))
    bt3 = bt.reshape(Npad // Rb, 1, Rb)
    stp = jnp.pad(state, ((0, 128 - B), (0, 0)))
    acp = jnp.pad(action, ((0, 128 - B), (0, 0)))
    q1w3p = jnp.pad(q1w3, ((0, 128 - q1w3.shape[0]), (0, 0)))
    q2w3p = jnp.pad(q2w3, ((0, 128 - q2w3.shape[0]), (0, 0)))
    qw = [q1w1, q1b1.reshape(1, -1), q1w2, q1b2.reshape(1, -1),
          q1w3p, q1b3.reshape(1, -1),
          q2w1, q2b1.reshape(1, -1), q2w2, q2b2.reshape(1, -1),
          q2w3p, q2b3.reshape(1, -1)]
    q1, q2 = _tc3(acc2, cnt, bt3, stp, acp, g2w2, g2b2.reshape(1, -1),
                  qw, Rb, B)
    return (q1, q2)
